# Initial kernel scaffold; baseline (speedup 1.0000x reference)
import jax
import jax.numpy as jnp
from jax.experimental import pallas as pl


def kernel(x, edge_index, W, b):
    raise NotImplementedError("write your pallas kernel here")


# speedup vs baseline: 44.1997x; 44.1997x over previous
import functools

import jax
import jax.numpy as jnp
from jax import lax
from jax.experimental import pallas as pl
from jax.experimental.pallas import tpu as pltpu
from jax.experimental.pallas import tpu_sc as plsc

N = 10000
E = 320000
D = 128
C = 16

NC = 2
NS = 16
NW = NC * NS
CH = 128
KJ = -(-E // (NW * CH))
EP = NW * KJ * CH
NP = 10112
RPT = NP // NS

_MESH = plsc.VectorSubcoreMesh(core_axis_name="c", subcore_axis_name="s")


@functools.partial(
    pl.kernel,
    out_type=jax.ShapeDtypeStruct((NC, NP, C), jnp.float32),
    mesh=_MESH,
    scratch_types=[
        pltpu.VMEM((KJ, CH), jnp.int32),
        pltpu.VMEM((CH, C), jnp.float32),
        pltpu.VMEM_SHARED((NP, C), jnp.float32),
    ],
)
def _sc_degree(cidx_hbm, zeros_hbm, ones_hbm, out_hbm, cidx_v, msg_v, acc_sh):
    c = lax.axis_index("c")
    s = lax.axis_index("s")
    w = s * NC + c
    pltpu.sync_copy(cidx_hbm.at[w], cidx_v)
    pltpu.sync_copy(ones_hbm, msg_v)
    rs = s * RPT
    pltpu.sync_copy(zeros_hbm.at[pl.ds(rs, RPT)], acc_sh.at[pl.ds(rs, RPT)])
    plsc.subcore_barrier()

    def body(j, carry):
        pltpu.sync_copy(msg_v, acc_sh.at[cidx_v.at[j]], add=True)
        return carry

    lax.fori_loop(0, KJ, body, 0)
    plsc.subcore_barrier()
    pltpu.sync_copy(acc_sh.at[pl.ds(rs, RPT)], out_hbm.at[c, pl.ds(rs, RPT)])


@functools.partial(
    pl.kernel,
    out_type=jax.ShapeDtypeStruct((NC, NP, C), jnp.float32),
    mesh=_MESH,
    scratch_types=[
        pltpu.VMEM((KJ, CH), jnp.int32),
        pltpu.VMEM((KJ, CH), jnp.int32),
        pltpu.VMEM((CH, C), jnp.float32),
        pltpu.VMEM_SHARED((NP, C), jnp.float32),
        pltpu.VMEM_SHARED((NP, C), jnp.float32),
        pltpu.SemaphoreType.DMA,
    ],
)
def _sc_hop(zs_hbm, ridx_hbm, cidx_hbm, zeros_hbm, out_hbm,
            ridx_v, cidx_v, msg_v, zs_sh, acc_sh, sem):
    c = lax.axis_index("c")
    s = lax.axis_index("s")
    w = s * NC + c
    pltpu.sync_copy(ridx_hbm.at[w], ridx_v)
    pltpu.sync_copy(cidx_hbm.at[w], cidx_v)
    rs = s * RPT
    pltpu.sync_copy(zs_hbm.at[pl.ds(rs, RPT)], zs_sh.at[pl.ds(rs, RPT)])
    pltpu.sync_copy(zeros_hbm.at[pl.ds(rs, RPT)], acc_sh.at[pl.ds(rs, RPT)])
    plsc.subcore_barrier()

    def body(j, carry):
        pltpu.async_copy(zs_sh.at[ridx_v.at[j]], msg_v, sem).wait()
        pltpu.sync_copy(msg_v, acc_sh.at[cidx_v.at[j]], add=True)
        return carry

    lax.fori_loop(0, KJ, body, 0)
    plsc.subcore_barrier()
    pltpu.sync_copy(acc_sh.at[pl.ds(rs, RPT)], out_hbm.at[c, pl.ds(rs, RPT)])


def _tc_matmul(x_ref, w_ref, o_ref):
    o_ref[...] = lax.dot_general(
        x_ref[...], w_ref[...], (((1,), (1,)), ((), ())),
        preferred_element_type=jnp.float32)


def _tc_prep(d0_ref, d1_ref, z_ref, zs_ref, disb_ref):
    deg = d0_ref[...] + d1_ref[...] + 1.0
    disb = lax.rsqrt(deg)
    disb_ref[...] = disb
    zs_ref[...] = z_ref[...] * disb


def _tc_mid(a0_ref, a1_ref, z_ref, disb_ref, h1_ref, zs2_ref):
    disb = disb_ref[...]
    h1 = disb * (a0_ref[...] + a1_ref[...]) + disb * disb * z_ref[...]
    h1_ref[...] = h1
    zs2_ref[...] = disb * h1


def _tc_final(a0_ref, a1_ref, h1_ref, disb_ref, b_ref, o_ref):
    disb = disb_ref[...]
    h2 = disb * (a0_ref[...] + a1_ref[...]) + disb * disb * h1_ref[...]
    o = h2 + b_ref[...]
    m = jnp.max(o, axis=1, keepdims=True)
    e = o - m
    o_ref[...] = e - jnp.log(jnp.sum(jnp.exp(e), axis=1, keepdims=True))


def kernel(x, edge_index, W, b):
    pad = EP - E
    row_p = jnp.concatenate([edge_index[0], jnp.zeros((pad,), jnp.int32)])
    col_p = jnp.concatenate([edge_index[1], jnp.full((pad,), N, jnp.int32)])
    ridx = row_p.reshape(NW, KJ, CH)
    cidx = col_p.reshape(NW, KJ, CH)
    x_p = jnp.concatenate([x, jnp.zeros((NP - N, D), x.dtype)])
    zeros_np = jnp.zeros((NP, C), jnp.float32)
    ones_ch = jnp.ones((CH, C), jnp.float32)

    degp = _sc_degree(cidx, zeros_np, ones_ch)

    z = pl.pallas_call(
        _tc_matmul,
        out_shape=jax.ShapeDtypeStruct((NP, C), jnp.float32),
    )(x_p, W)

    zs1, disb = pl.pallas_call(
        _tc_prep,
        out_shape=(jax.ShapeDtypeStruct((NP, C), jnp.float32),
                   jax.ShapeDtypeStruct((NP, C), jnp.float32)),
    )(degp[0], degp[1], z)

    acc1 = _sc_hop(zs1, ridx, cidx, zeros_np)

    h1, zs2 = pl.pallas_call(
        _tc_mid,
        out_shape=(jax.ShapeDtypeStruct((NP, C), jnp.float32),
                   jax.ShapeDtypeStruct((NP, C), jnp.float32)),
    )(acc1[0], acc1[1], z, disb)

    acc2 = _sc_hop(zs2, ridx, cidx, zeros_np)

    out = pl.pallas_call(
        _tc_final,
        out_shape=jax.ShapeDtypeStruct((NP, C), jnp.float32),
    )(acc2[0], acc2[1], h1, disb, b.reshape(1, C))
    return out[:N]
